# baseline (speedup 1.0000x reference)
import functools

import jax
import jax.numpy as jnp
from jax import lax
from jax.experimental import pallas as pl
from jax.experimental.pallas import tpu as pltpu
from jax.experimental.pallas import tpu_sc as plsc

EMB = 128
VOCAB1 = 100001
UTT = 20
BATCH = 4096
NMT = 26
MPT = 64
NOUT = NMT * MPT

NC = 2
NS = 16
NW = NC * NS
PERW = BATCH // NW
NB = 16
NCH = PERW // NB
PU = 5
QN = UTT // PU
GR = UT_ROWS = PU * NB
NU = NCH * QN
K = 4
LANES = 16


def _sc_body(utts_hbm, table_hbm, embs_hbm, idx_v, idx_c, out_v,
             bufs, sems):
    wid = lax.axis_index("s") * NC + lax.axis_index("c")
    base = wid * PERW

    pltpu.sync_copy(utts_hbm.at[:, pl.ds(base, PERW)], idx_v)
    for ci in range(NCH):
        for p in range(UTT):
            flat = ci * UTT * NB + (p // PU) * GR + (p % PU) * NB
            idx_c[pl.ds(flat, LANES)] = (
                idx_v[p, pl.ds(ci * NB, LANES)] + (p * VOCAB1)
            )

    def fire(u, buf, sem):
        pltpu.async_copy(table_hbm.at[idx_c.at[pl.ds(u * GR, GR)]], buf, sem)

    def drain(u, buf, sem):
        pltpu.make_async_copy(
            table_hbm.at[idx_c.at[pl.ds(u * GR, GR)]], buf, sem).wait()

    def sum_unit(ci, q, buf):
        def bbody(b, c2):
            for c in range(EMB // LANES):
                sl = pl.ds(c * LANES, LANES)
                acc = buf[b, sl]
                for pp in range(1, PU):
                    acc = acc + buf[pp * NB + b, sl]
                if q == 0:
                    out_v[ci * NB + b, sl] = acc
                else:
                    out_v[ci * NB + b, sl] = out_v[ci * NB + b, sl] + acc
            return c2

        lax.fori_loop(0, NB, bbody, 0)

    for k in range(K - 1):
        fire(k, bufs[k], sems[k])

    def ring_body(i, carry):
        for k in range(K):
            u = i * K + k
            drain(u, bufs[k], sems[k])
            sum_unit(i, k, bufs[k])

            @pl.when(u + (K - 1) < NU)
            def _():
                fire(u + (K - 1), bufs[(k + K - 1) % K], sems[(k + K - 1) % K])
        return carry

    lax.fori_loop(0, NU // K, ring_body, 0)
    pltpu.sync_copy(out_v, embs_hbm.at[pl.ds(base, PERW)])


@functools.partial(
    pl.kernel,
    mesh=plsc.VectorSubcoreMesh(core_axis_name="c", subcore_axis_name="s"),
    out_type=jax.ShapeDtypeStruct((BATCH, EMB), jnp.float32),
    scratch_types=[
        pltpu.VMEM((UTT, PERW), jnp.int32),
        pltpu.VMEM((NU * UT_ROWS,), jnp.int32),
        pltpu.VMEM((PERW, EMB), jnp.float32),
        pltpu.VMEM((UT_ROWS, EMB), jnp.float32),
        pltpu.VMEM((UT_ROWS, EMB), jnp.float32),
        pltpu.VMEM((UT_ROWS, EMB), jnp.float32),
        pltpu.VMEM((UT_ROWS, EMB), jnp.float32),
        pltpu.SemaphoreType.DMA,
        pltpu.SemaphoreType.DMA,
        pltpu.SemaphoreType.DMA,
        pltpu.SemaphoreType.DMA,
    ],
)
def _sc_gather_sum(utts_hbm, table_hbm, embs_hbm, idx_v, idx_c, out_v,
                   b0, b1, b2, b3, s0, s1, s2, s3):
    _sc_body(utts_hbm, table_hbm, embs_hbm, idx_v, idx_c, out_v,
             [b0, b1, b2, b3], [s0, s1, s2, s3])


def _tc_body(e_ref, w_ref, b_ref, o_ref):
    x = jnp.tanh(e_ref[...])
    y = lax.dot_general(
        w_ref[...], x, (((1,), (1,)), ((), ())),
        preferred_element_type=jnp.float32)
    o_ref[...] = y + b_ref[...]


_TB = 512


def _tc_dense_t(embs, W2t, b2col):
    return pl.pallas_call(
        _tc_body,
        grid=(BATCH // _TB,),
        in_specs=[
            pl.BlockSpec((_TB, EMB), lambda i: (i, 0)),
            pl.BlockSpec((NOUT, EMB), lambda i: (0, 0)),
            pl.BlockSpec((NOUT, 1), lambda i: (0, 0)),
        ],
        out_specs=pl.BlockSpec((NOUT, _TB), lambda i: (0, i)),
        out_shape=jax.ShapeDtypeStruct((NOUT, BATCH), jnp.float32),
    )(embs, W2t, b2col)


def kernel(utts, emb_table, W2, b2):
    embs = _sc_gather_sum(utts, emb_table)
    yt = _tc_dense_t(embs, W2.T, b2.reshape(NOUT, 1))
    return yt.reshape(NMT, MPT, BATCH).transpose(2, 0, 1)
